# baseline (speedup 1.0000x reference)
import functools

import jax
import jax.numpy as jnp
from jax import lax
from jax.experimental import pallas as pl
from jax.experimental.pallas import tpu as pltpu
from jax.experimental.pallas import tpu_sc as plsc

DIM = 16
L = 16
NC, NS = 2, 16
NW = NC * NS


def _rsqrt_newton(s):
    i = lax.bitcast_convert_type(s, jnp.int32)
    i = jnp.int32(0x5F3759DF) - lax.shift_right_arithmetic(i, 1)
    y = lax.bitcast_convert_type(i, jnp.float32)
    for _ in range(3):
        y = y * (jnp.float32(1.5) - jnp.float32(0.5) * s * y * y)
    return y


def _make_sc_loss(batch):
    bpw = batch // NW
    nidx = 2 * bpw
    nelem = nidx * DIM
    ngrp = bpw // L
    mesh = plsc.VectorSubcoreMesh(core_axis_name="c", subcore_axis_name="s")

    @functools.partial(
        pl.kernel,
        mesh=mesh,
        out_type=jax.ShapeDtypeStruct((NW, L), jnp.float32),
        scratch_types=[
            pltpu.VMEM((nidx,), jnp.int32),
            pltpu.VMEM((nelem,), jnp.int32),
            pltpu.VMEM((nelem,), jnp.float32),
            pltpu.VMEM((bpw,), jnp.float32),
            pltpu.VMEM((L,), jnp.float32),
            pltpu.SemaphoreType.DMA,
        ],
        compiler_params=pltpu.CompilerParams(needs_layout_passes=False),
    )
    def sc_loss(x_hbm, deg_hbm, emb_hbm, out_hbm, idx_v, idx16_v, elems_v,
                deg_v, acc_v, sem):
        wid = lax.axis_index("s") * NC + lax.axis_index("c")
        base = wid * bpw
        pltpu.sync_copy(x_hbm.at[pl.ds(2 * base, nidx)], idx_v)
        pltpu.sync_copy(deg_hbm.at[pl.ds(base, bpw)], deg_v)

        lane = lax.iota(jnp.int32, L)

        def expand_body(j, carry):
            ebase = lax.shift_left(idx_v[pl.ds(j * L, L)], 4)
            pos = j * (L * DIM) + lane * DIM
            for d in range(DIM):
                plsc.store_scatter(idx16_v, [pos + d], ebase + d)
            return carry

        lax.fori_loop(0, nidx // L, expand_body, jnp.int32(0))

        pltpu.async_copy(emb_hbm.at[idx16_v], elems_v, sem).wait()

        def grp_body(g, acc):
            even = (2 * (g * L + lane)) * DIM
            odd = even + DIM
            ssum = jnp.zeros((L,), jnp.float32)
            for d in range(DIM):
                a = plsc.load_gather(elems_v, [even + d])
                b = plsc.load_gather(elems_v, [odd + d])
                diff = a - b
                ssum = ssum + diff * diff
            ssum = jnp.maximum(ssum, jnp.float32(1e-30))
            dist = ssum * _rsqrt_newton(ssum)
            rate = jnp.exp(-dist)
            deg = deg_v[pl.ds(g * L, L)]
            return acc + deg * dist + rate

        acc = lax.fori_loop(0, ngrp, grp_body, jnp.zeros((L,), jnp.float32))
        acc_v[...] = acc
        pltpu.sync_copy(acc_v, out_hbm.at[wid])

    return sc_loss


@jax.jit
def kernel(x, degrees, embs):
    batch = x.shape[0]
    x_flat = x.astype(jnp.int32).reshape(-1)
    emb_flat = embs.reshape(-1)
    partials = _make_sc_loss(batch)(x_flat, degrees, emb_flat)
    return jnp.sum(partials)


# speedup vs baseline: 1.0199x; 1.0199x over previous
import functools

import jax
import jax.numpy as jnp
from jax import lax
from jax.experimental import pallas as pl
from jax.experimental.pallas import tpu as pltpu
from jax.experimental.pallas import tpu_sc as plsc

DIM = 16
L = 16
NC, NS = 2, 16
NW = NC * NS


def _rsqrt_newton(s):
    i = lax.bitcast_convert_type(s, jnp.int32)
    i = jnp.int32(0x5F3759DF) - lax.shift_right_arithmetic(i, 1)
    y = lax.bitcast_convert_type(i, jnp.float32)
    for _ in range(3):
        y = y * (jnp.float32(1.5) - jnp.float32(0.5) * s * y * y)
    return y


def _make_sc_loss(batch):
    bpw = batch // NW
    nidx = 2 * bpw
    nelem = nidx * DIM
    ngrp = bpw // L
    mesh = plsc.VectorSubcoreMesh(core_axis_name="c", subcore_axis_name="s")

    @functools.partial(
        pl.kernel,
        mesh=mesh,
        out_type=jax.ShapeDtypeStruct((NW, L), jnp.float32),
        scratch_types=[
            pltpu.VMEM((nidx,), jnp.int32),
            pltpu.VMEM((nelem,), jnp.int32),
            pltpu.VMEM((nelem,), jnp.float32),
            pltpu.VMEM((bpw,), jnp.float32),
            pltpu.VMEM((L,), jnp.float32),
            pltpu.SemaphoreType.DMA,
        ],
        compiler_params=pltpu.CompilerParams(needs_layout_passes=False),
    )
    def sc_loss(x_hbm, deg_hbm, emb_hbm, out_hbm, idx_v, idx16_v, elems_v,
                deg_v, acc_v, sem):
        wid = lax.axis_index("s") * NC + lax.axis_index("c")
        base = wid * bpw
        pltpu.sync_copy(x_hbm.at[pl.ds(2 * base, nidx)], idx_v)
        pltpu.sync_copy(deg_hbm.at[pl.ds(base, bpw)], deg_v)

        def expand_body(j, carry):
            idx = idx_v[pl.ds(j * L, L)]
            ebase = lax.shift_left(idx, 4)
            for d in range(DIM):
                idx16_v[pl.ds(d * nidx + j * L, L)] = ebase + d
            return carry

        lax.fori_loop(0, nidx // L, expand_body, jnp.int32(0))

        pltpu.async_copy(emb_hbm.at[idx16_v], elems_v, sem).wait()

        lane = lax.iota(jnp.int32, L)

        def grp_body(g, acc):
            even = 2 * (g * L + lane)
            ssum = jnp.zeros((L,), jnp.float32)
            for d in range(DIM):
                a = plsc.load_gather(elems_v, [d * nidx + even])
                b = plsc.load_gather(elems_v, [d * nidx + even + 1])
                diff = a - b
                ssum = ssum + diff * diff
            ssum = jnp.maximum(ssum, jnp.float32(1e-30))
            dist = ssum * _rsqrt_newton(ssum)
            rate = jnp.exp(-dist)
            deg = deg_v[pl.ds(g * L, L)]
            return acc + deg * dist + rate

        acc = lax.fori_loop(0, ngrp, grp_body, jnp.zeros((L,), jnp.float32))
        acc_v[...] = acc
        pltpu.sync_copy(acc_v, out_hbm.at[wid])

    return sc_loss


@jax.jit
def kernel(x, degrees, embs):
    batch = x.shape[0]
    x_flat = x.astype(jnp.int32).reshape(-1)
    emb_flat = embs.reshape(-1)
    partials = _make_sc_loss(batch)(x_flat, degrees, emb_flat)
    return jnp.sum(partials)
